# baseline (speedup 1.0000x reference)
import jax
import jax.numpy as jnp
from jax.experimental import pallas as pl
from jax.experimental.pallas import tpu as pltpu

_NU, _NV, _NC, _D, _H = 943, 1682, 5, 64, 32
_BU = 256
_GU = (_NU + _BU - 1) // _BU
_NUP = _GU * _BU


def _split16(x):
    hi = x.astype(jnp.bfloat16)
    lo = (x - hi.astype(jnp.float32)).astype(jnp.bfloat16)
    return hi, lo


def _fused_kernel(r_ref, nu_ref, nv_ref, nvt_ref, uf_ref, vf_ref, wu_ref,
                  wv_ref, bu_ref, bv_ref, q_ref,
                  out_ref, loss_ref, acc_ref,
                  sums, tv_hi_ref, tv_lo_ref, u2_ref, v2t_ref, code_ref):
    p = pl.program_id(0)
    i = pl.program_id(1)
    sl = pl.ds(i * _BU, _BU)

    def _gconv_body(masked):
        cu_col = jax.lax.rsqrt(jnp.maximum(nu_ref[...], 1.0))
        cv_row = jax.lax.rsqrt(jnp.maximum(nv_ref[...], 1.0))
        cvt_col = jax.lax.rsqrt(jnp.maximum(nvt_ref[...], 1.0))

        ufb = uf_ref[...]
        if masked:
            rows = jax.lax.broadcasted_iota(jnp.int32, (_BU, 1), 0) + i * _BU
            row_ok = rows < _NU
            ufb = jnp.where(row_ok, ufb, 0.0)

        @pl.when(i == 0)
        def _():
            for k in range(_NC):
                t_v = jnp.dot(vf_ref[...], wv_ref[k],
                              preferred_element_type=jnp.float32)
                hi, lo = _split16(t_v * cvt_col)
                tv_hi_ref[k] = hi
                tv_lo_ref[k] = lo

        rblk = r_ref[...]

        ucontrib = jnp.zeros((_BU, _H), jnp.float32)
        vcontribT = jnp.zeros((_H, _NV), jnp.float32)
        code_f = jnp.zeros((_BU, _NV), jnp.float32)
        for k in range(_NC):
            rkm = jnp.where(row_ok, rblk[k], 0.0) if masked else rblk[k]
            code_f = code_f + rkm * float(k + 1)
            rk16 = rkm.astype(jnp.bfloat16)
            t_u = jnp.dot(ufb, wu_ref[k], preferred_element_type=jnp.float32)
            tu_hi, tu_lo = _split16(t_u * cu_col)
            ucontrib = (
                ucontrib
                + jnp.dot(rk16, tv_hi_ref[k],
                          preferred_element_type=jnp.float32)
                + jnp.dot(rk16, tv_lo_ref[k],
                          preferred_element_type=jnp.float32))
            vcontribT = (
                vcontribT
                + jax.lax.dot_general(
                    tu_hi, rk16, (((0,), (0,)), ((), ())),
                    preferred_element_type=jnp.float32)
                + jax.lax.dot_general(
                    tu_lo, rk16, (((0,), (0,)), ((), ())),
                    preferred_element_type=jnp.float32))

        code_ref[sl, :] = code_f.astype(jnp.int8)

        u2_ref[sl, :] = jnp.maximum(ucontrib * cu_col + bu_ref[...], 0.0)

        @pl.when(i == 0)
        def _():
            v2t_ref[...] = vcontribT

        @pl.when(i > 0)
        def _():
            v2t_ref[...] = v2t_ref[...] + vcontribT

        @pl.when(i == _GU - 1)
        def _():
            v2t_ref[...] = jnp.maximum(v2t_ref[...] * cv_row + bv_ref[...],
                                       0.0)

    @pl.when(jnp.logical_and(p == 0, i < _GU - 1))
    def _():
        _gconv_body(False)

    @pl.when(jnp.logical_and(p == 0, i == _GU - 1))
    def _():
        _gconv_body(True)

    @pl.when(p == 1)
    def _decode():
        @pl.when(i == 0)
        def _():
            sums[0] = 0.0
            sums[1] = 0.0
            sums[2] = 0.0

        u2b = u2_ref[sl, :]
        v2tb = v2t_ref[...]
        code = code_ref[sl, :].astype(jnp.int32)
        rated = code > 0
        tcls = code - 1

        zs = []
        for k in range(_NC):
            uq = jnp.dot(u2b, q_ref[k], preferred_element_type=jnp.float32)
            z = jnp.dot(uq, v2tb, preferred_element_type=jnp.float32)
            out_ref[k] = z
            zs.append(z)

        m = zs[0]
        pred = jnp.zeros((_BU, _NV), jnp.int32)
        for k in range(1, _NC):
            gt = zs[k] > m
            pred = jnp.where(gt, k, pred)
            m = jnp.maximum(m, zs[k])
        s = jnp.zeros((_BU, _NV), jnp.float32)
        for k in range(_NC):
            s = s + jnp.exp(zs[k] - m)
        lse = m + jnp.log(s)

        ztrue = jnp.zeros((_BU, _NV), jnp.float32)
        for k in range(_NC):
            ztrue = jnp.where(tcls == k, zs[k], ztrue)

        loss_c = jnp.sum(jnp.where(rated, ztrue - lse, 0.0))
        mask_c = jnp.sum(jnp.where(rated, 1.0, 0.0))
        corr_c = jnp.sum(
            jnp.where(jnp.logical_and(rated, pred == tcls), 1.0, 0.0))
        sums[0] = sums[0] + loss_c
        sums[1] = sums[1] + mask_c
        sums[2] = sums[2] + corr_c

        @pl.when(i == _GU - 1)
        def _():
            denom = jnp.maximum(sums[1], 1.0)
            loss_ref[...] = jnp.full((1, 1), -sums[0] / denom, jnp.float32)
            acc_ref[...] = jnp.full((1, 1), sums[2] / denom, jnp.float32)


def kernel(u, v, r, n, c, u_emb_w, v_emb_w, Wu1, Wv1, bu1, bv1,
           Wu2, Wv2, bu2, bv2, Q):
    uf = jnp.take(u_emb_w, u, axis=0)
    vf = jnp.take(v_emb_w, v, axis=0)
    nu = jnp.pad(n[:_NU].reshape(_NU, 1), ((0, _NUP - _NU), (0, 0)),
                 constant_values=1.0)
    nv_row = n[_NU:].reshape(1, _NV)
    nv_col = n[_NU:].reshape(_NV, 1)

    def _imap(p, i):
        return i * (1 - p) + (_GU - 1) * p

    outputs, lossm, accm = pl.pallas_call(
        _fused_kernel,
        grid=(2, _GU),
        in_specs=[
            pl.BlockSpec((_NC, _BU, _NV), lambda p, i: (0, _imap(p, i), 0)),
            pl.BlockSpec((_BU, 1), lambda p, i: (_imap(p, i), 0)),
            pl.BlockSpec((1, _NV), lambda p, i: (0, 0)),
            pl.BlockSpec((_NV, 1), lambda p, i: (0, 0)),
            pl.BlockSpec((_BU, _D), lambda p, i: (_imap(p, i), 0)),
            pl.BlockSpec((_NV, _D), lambda p, i: (0, 0)),
            pl.BlockSpec((_NC, _D, _H), lambda p, i: (0, 0, 0)),
            pl.BlockSpec((_NC, _D, _H), lambda p, i: (0, 0, 0)),
            pl.BlockSpec((1, _H), lambda p, i: (0, 0)),
            pl.BlockSpec((_H, 1), lambda p, i: (0, 0)),
            pl.BlockSpec((_NC, _H, _H), lambda p, i: (0, 0, 0)),
        ],
        out_specs=[
            pl.BlockSpec((_NC, _BU, _NV), lambda p, i: (0, i * p, 0)),
            pl.BlockSpec((1, 1), lambda p, i: (0, 0)),
            pl.BlockSpec((1, 1), lambda p, i: (0, 0)),
        ],
        out_shape=[
            jax.ShapeDtypeStruct((_NC, _NU, _NV), jnp.float32),
            jax.ShapeDtypeStruct((1, 1), jnp.float32),
            jax.ShapeDtypeStruct((1, 1), jnp.float32),
        ],
        scratch_shapes=[
            pltpu.SMEM((4,), jnp.float32),
            pltpu.VMEM((_NC, _NV, _H), jnp.bfloat16),
            pltpu.VMEM((_NC, _NV, _H), jnp.bfloat16),
            pltpu.VMEM((_NUP, _H), jnp.float32),
            pltpu.VMEM((_H, _NV), jnp.float32),
            pltpu.VMEM((_NUP, _NV), jnp.int8),
        ],
        compiler_params=pltpu.CompilerParams(
            dimension_semantics=("arbitrary", "arbitrary")),
    )(r, nu, nv_row, nv_col, uf, vf, Wu2, Wv2,
      bu2.reshape(1, _H), bv2.reshape(_H, 1), Q)

    return outputs, lossm[0, 0], accm[0, 0]


# speedup vs baseline: 1.0184x; 1.0184x over previous
import jax
import jax.numpy as jnp
from jax.experimental import pallas as pl
from jax.experimental.pallas import tpu as pltpu

_NU, _NV, _NC, _D, _H = 943, 1682, 5, 64, 32
_BU = 192
_GU = (_NU + _BU - 1) // _BU
_NUP = _GU * _BU


def _split16(x):
    hi = x.astype(jnp.bfloat16)
    lo = (x - hi.astype(jnp.float32)).astype(jnp.bfloat16)
    return hi, lo


def _fused_kernel(r_ref, nu_ref, nv_ref, nvt_ref, uf_ref, vf_ref, wu_ref,
                  wv_ref, bu_ref, bv_ref, q_ref,
                  out_ref, loss_ref, acc_ref,
                  sums, tv_hi_ref, tv_lo_ref, u2_ref, v2t_ref, code_ref):
    p = pl.program_id(0)
    i = pl.program_id(1)
    sl = pl.ds(i * _BU, _BU)

    def _gconv_body(masked):
        cu_col = jax.lax.rsqrt(jnp.maximum(nu_ref[...], 1.0))
        cv_row = jax.lax.rsqrt(jnp.maximum(nv_ref[...], 1.0))
        cvt_col = jax.lax.rsqrt(jnp.maximum(nvt_ref[...], 1.0))

        ufb = uf_ref[...]
        if masked:
            rows = jax.lax.broadcasted_iota(jnp.int32, (_BU, 1), 0) + i * _BU
            row_ok = rows < _NU
            ufb = jnp.where(row_ok, ufb, 0.0)

        @pl.when(i == 0)
        def _():
            for k in range(_NC):
                t_v = jnp.dot(vf_ref[...], wv_ref[k],
                              preferred_element_type=jnp.float32)
                hi, lo = _split16(t_v * cvt_col)
                tv_hi_ref[k] = hi
                tv_lo_ref[k] = lo

        rblk = r_ref[...]

        ucontrib = jnp.zeros((_BU, _H), jnp.float32)
        vcontribT = jnp.zeros((_H, _NV), jnp.float32)
        code_f = jnp.zeros((_BU, _NV), jnp.float32)
        for k in range(_NC):
            rkm = jnp.where(row_ok, rblk[k], 0.0) if masked else rblk[k]
            code_f = code_f + rkm * float(k + 1)
            rk16 = rkm.astype(jnp.bfloat16)
            t_u = jnp.dot(ufb, wu_ref[k], preferred_element_type=jnp.float32)
            tu_hi, tu_lo = _split16(t_u * cu_col)
            ucontrib = (
                ucontrib
                + jnp.dot(rk16, tv_hi_ref[k],
                          preferred_element_type=jnp.float32)
                + jnp.dot(rk16, tv_lo_ref[k],
                          preferred_element_type=jnp.float32))
            vcontribT = (
                vcontribT
                + jax.lax.dot_general(
                    tu_hi, rk16, (((0,), (0,)), ((), ())),
                    preferred_element_type=jnp.float32)
                + jax.lax.dot_general(
                    tu_lo, rk16, (((0,), (0,)), ((), ())),
                    preferred_element_type=jnp.float32))

        code_ref[sl, :] = code_f.astype(jnp.int8)

        u2_ref[sl, :] = jnp.maximum(ucontrib * cu_col + bu_ref[...], 0.0)

        @pl.when(i == 0)
        def _():
            v2t_ref[...] = vcontribT

        @pl.when(i > 0)
        def _():
            v2t_ref[...] = v2t_ref[...] + vcontribT

        @pl.when(i == _GU - 1)
        def _():
            v2t_ref[...] = jnp.maximum(v2t_ref[...] * cv_row + bv_ref[...],
                                       0.0)

    @pl.when(jnp.logical_and(p == 0, i < _GU - 1))
    def _():
        _gconv_body(False)

    @pl.when(jnp.logical_and(p == 0, i == _GU - 1))
    def _():
        _gconv_body(True)

    @pl.when(p == 1)
    def _decode():
        @pl.when(i == 0)
        def _():
            sums[0] = 0.0
            sums[1] = 0.0
            sums[2] = 0.0

        u2b = u2_ref[sl, :]
        v2tb = v2t_ref[...]
        code = code_ref[sl, :].astype(jnp.int32)
        rated = code > 0
        tcls = code - 1

        zs = []
        for k in range(_NC):
            uq = jnp.dot(u2b, q_ref[k], preferred_element_type=jnp.float32)
            z = jnp.dot(uq, v2tb, preferred_element_type=jnp.float32)
            out_ref[k] = z
            zs.append(z)

        m = zs[0]
        pred = jnp.zeros((_BU, _NV), jnp.int32)
        for k in range(1, _NC):
            gt = zs[k] > m
            pred = jnp.where(gt, k, pred)
            m = jnp.maximum(m, zs[k])
        s = jnp.zeros((_BU, _NV), jnp.float32)
        for k in range(_NC):
            s = s + jnp.exp(zs[k] - m)
        lse = m + jnp.log(s)

        ztrue = jnp.zeros((_BU, _NV), jnp.float32)
        for k in range(_NC):
            ztrue = jnp.where(tcls == k, zs[k], ztrue)

        loss_c = jnp.sum(jnp.where(rated, ztrue - lse, 0.0))
        mask_c = jnp.sum(jnp.where(rated, 1.0, 0.0))
        corr_c = jnp.sum(
            jnp.where(jnp.logical_and(rated, pred == tcls), 1.0, 0.0))
        sums[0] = sums[0] + loss_c
        sums[1] = sums[1] + mask_c
        sums[2] = sums[2] + corr_c

        @pl.when(i == _GU - 1)
        def _():
            denom = jnp.maximum(sums[1], 1.0)
            loss_ref[...] = jnp.full((1, 1), -sums[0] / denom, jnp.float32)
            acc_ref[...] = jnp.full((1, 1), sums[2] / denom, jnp.float32)


def kernel(u, v, r, n, c, u_emb_w, v_emb_w, Wu1, Wv1, bu1, bv1,
           Wu2, Wv2, bu2, bv2, Q):
    uf = jnp.take(u_emb_w, u, axis=0)
    vf = jnp.take(v_emb_w, v, axis=0)
    nu = jnp.pad(n[:_NU].reshape(_NU, 1), ((0, _NUP - _NU), (0, 0)),
                 constant_values=1.0)
    nv_row = n[_NU:].reshape(1, _NV)
    nv_col = n[_NU:].reshape(_NV, 1)

    def _imap(p, i):
        return i * (1 - p) + (_GU - 1) * p

    outputs, lossm, accm = pl.pallas_call(
        _fused_kernel,
        grid=(2, _GU),
        in_specs=[
            pl.BlockSpec((_NC, _BU, _NV), lambda p, i: (0, _imap(p, i), 0)),
            pl.BlockSpec((_BU, 1), lambda p, i: (_imap(p, i), 0)),
            pl.BlockSpec((1, _NV), lambda p, i: (0, 0)),
            pl.BlockSpec((_NV, 1), lambda p, i: (0, 0)),
            pl.BlockSpec((_BU, _D), lambda p, i: (_imap(p, i), 0)),
            pl.BlockSpec((_NV, _D), lambda p, i: (0, 0)),
            pl.BlockSpec((_NC, _D, _H), lambda p, i: (0, 0, 0)),
            pl.BlockSpec((_NC, _D, _H), lambda p, i: (0, 0, 0)),
            pl.BlockSpec((1, _H), lambda p, i: (0, 0)),
            pl.BlockSpec((_H, 1), lambda p, i: (0, 0)),
            pl.BlockSpec((_NC, _H, _H), lambda p, i: (0, 0, 0)),
        ],
        out_specs=[
            pl.BlockSpec((_NC, _BU, _NV), lambda p, i: (0, i * p, 0)),
            pl.BlockSpec((1, 1), lambda p, i: (0, 0)),
            pl.BlockSpec((1, 1), lambda p, i: (0, 0)),
        ],
        out_shape=[
            jax.ShapeDtypeStruct((_NC, _NU, _NV), jnp.float32),
            jax.ShapeDtypeStruct((1, 1), jnp.float32),
            jax.ShapeDtypeStruct((1, 1), jnp.float32),
        ],
        scratch_shapes=[
            pltpu.SMEM((4,), jnp.float32),
            pltpu.VMEM((_NC, _NV, _H), jnp.bfloat16),
            pltpu.VMEM((_NC, _NV, _H), jnp.bfloat16),
            pltpu.VMEM((_NUP, _H), jnp.float32),
            pltpu.VMEM((_H, _NV), jnp.float32),
            pltpu.VMEM((_NUP, _NV), jnp.int8),
        ],
        compiler_params=pltpu.CompilerParams(
            dimension_semantics=("arbitrary", "arbitrary")),
    )(r, nu, nv_row, nv_col, uf, vf, Wu2, Wv2,
      bu2.reshape(1, _H), bv2.reshape(_H, 1), Q)

    return outputs, lossm[0, 0], accm[0, 0]
